# baseline (speedup 1.0000x reference)
import functools

import jax
import jax.numpy as jnp
from jax import lax
from jax.experimental import pallas as pl
from jax.experimental.pallas import tpu as pltpu
from jax.experimental.pallas import tpu_sc as plsc

_NC = 2
_NS = 16
_NW = _NC * _NS
_G = 128


def _point_indices(points, s0, s1):
    _, P, _ = points.shape
    pt = 16384

    def body(p_ref, out_ref):
        p = p_ref[...]
        ix = jnp.floor(p[0, :, 0:1]).astype(jnp.int32)
        iy = jnp.floor(p[0, :, 1:2]).astype(jnp.int32)
        iz = jnp.floor(p[0, :, 2:3]).astype(jnp.int32)
        ind = ix * s0 + iy * s1 + iz
        out_ref[...] = ind.reshape(pt // 128, 128)

    return pl.pallas_call(
        body,
        grid=(P // pt,),
        in_specs=[pl.BlockSpec((1, pt, 3), lambda i: (0, i, 0))],
        out_specs=pl.BlockSpec((pt // 128, 128), lambda i: (i, 0)),
        out_shape=jax.ShapeDtypeStruct((P // 128, 128), jnp.int32),
    )(points)


def _mlp_table(h, w1, b1, w2p, b2p):
    C = h.shape[1]
    d0, d1, d2 = h.shape[2], h.shape[3], h.shape[4]
    V = d0 * d1 * d2
    vt = d1 * d2
    H = w1.shape[0]
    OP = w2p.shape[0]

    def body(h_ref, w1_ref, b1_ref, w2_ref, b2_ref, out_ref):
        blk = h_ref[...].reshape(C, vt).astype(jnp.bfloat16)
        x1 = lax.dot_general(
            blk, w1_ref[...].astype(jnp.bfloat16), (((0,), (1,)), ((), ())),
            preferred_element_type=jnp.float32)
        x1 = jnp.maximum(x1 + b1_ref[...], 0.0).astype(jnp.bfloat16)
        x2 = lax.dot_general(
            x1, w2_ref[...].astype(jnp.bfloat16), (((1,), (1,)), ((), ())),
            preferred_element_type=jnp.float32)
        out_ref[...] = x2 + b2_ref[...]

    return pl.pallas_call(
        body,
        grid=(d0,),
        in_specs=[
            pl.BlockSpec((1, C, 1, d1, d2), lambda i: (0, 0, i, 0, 0)),
            pl.BlockSpec((H, C), lambda i: (0, 0)),
            pl.BlockSpec((1, H), lambda i: (0, 0)),
            pl.BlockSpec((OP, H), lambda i: (0, 0)),
            pl.BlockSpec((1, OP), lambda i: (0, 0)),
        ],
        out_specs=pl.BlockSpec((vt, OP), lambda i: (i, 0)),
        out_shape=jax.ShapeDtypeStruct((V, OP), jnp.float32),
    )(h, w1, b1, w2p, b2p)


def _gather_rows(table, idx2d, n_out):
    V, OP = table.shape
    NR, G = idx2d.shape
    P = NR * G
    npw = P // _NW
    nchunk = npw // G
    mesh = plsc.VectorSubcoreMesh(core_axis_name="c", subcore_axis_name="s")

    @functools.partial(
        pl.kernel,
        mesh=mesh,
        compiler_params=pltpu.CompilerParams(use_tc_tiling_on_sc=False),
        out_type=jax.ShapeDtypeStruct((P, n_out), jnp.float32),
        scratch_types=[
            pltpu.VMEM((nchunk, G), jnp.int32),
            pltpu.VMEM((2, G, OP), jnp.float32),
            pltpu.SemaphoreType.DMA,
            pltpu.SemaphoreType.DMA,
        ],
    )
    def body(table_hbm, idx_hbm, out_hbm, idx_v, row_v, sem0, sem1):
        wid = lax.axis_index("s") * _NC + lax.axis_index("c")
        base = wid * npw
        pltpu.sync_copy(idx_hbm.at[pl.ds(wid * nchunk, nchunk)], idx_v)

        sems = (sem0, sem1)

        def start_gather(j, b):
            pltpu.async_copy(
                table_hbm.at[idx_v.at[j]], row_v.at[b], sems[b])

        def drain_write(j, b):
            pltpu.make_async_copy(
                table_hbm.at[idx_v.at[j]], row_v.at[b], sems[b]).wait()
            pltpu.sync_copy(
                row_v.at[b, :, pl.ds(0, n_out)],
                out_hbm.at[pl.ds(base + j * G, G)])

        start_gather(0, 0)

        def gather_body(jj, carry):
            j0 = jj * 2

            @pl.when(j0 + 1 < nchunk)
            def _():
                start_gather(j0 + 1, 1)

            drain_write(j0, 0)

            @pl.when(j0 + 1 < nchunk)
            def _():
                @pl.when(j0 + 2 < nchunk)
                def _():
                    start_gather(j0 + 2, 0)

                drain_write(j0 + 1, 1)

            return carry

        lax.fori_loop(0, (nchunk + 1) // 2, gather_body, 0)

    return body(table, idx2d)


def kernel(h, points, W1, b1, W2, b2):
    C = h.shape[1]
    d1, d2 = h.shape[3], h.shape[4]
    P = points.shape[1]
    H = W1.shape[0]
    OUT = W2.shape[0]
    OP = 128

    w2p = jnp.zeros((OP, H), W2.dtype).at[:OUT, :].set(W2)
    b2p = jnp.zeros((OP,), b2.dtype).at[:OUT].set(b2)

    idx2d = _point_indices(points, d1 * d2, d2)
    table = _mlp_table(h, W1, b1.reshape(1, H), w2p, b2p.reshape(1, OP))
    n_out = ((OUT + 7) // 8) * 8
    out = _gather_rows(table, idx2d, n_out)
    return out[:, :OUT]


# speedup vs baseline: 1.2405x; 1.2405x over previous
import functools

import jax
import jax.numpy as jnp
from jax import lax
from jax.experimental import pallas as pl
from jax.experimental.pallas import tpu as pltpu
from jax.experimental.pallas import tpu_sc as plsc

_NC = 2
_NS = 16
_NW = _NC * _NS
_G = 128


def _point_indices(points, s0, s1):
    _, P, _ = points.shape
    pt = 16384

    def body(p_ref, out_ref):
        p = p_ref[...]
        ix = jnp.floor(p[0, :, 0:1]).astype(jnp.int32)
        iy = jnp.floor(p[0, :, 1:2]).astype(jnp.int32)
        iz = jnp.floor(p[0, :, 2:3]).astype(jnp.int32)
        ind = ix * s0 + iy * s1 + iz
        out_ref[...] = ind.reshape(pt // 128, 128)

    return pl.pallas_call(
        body,
        grid=(P // pt,),
        in_specs=[pl.BlockSpec((1, pt, 3), lambda i: (0, i, 0))],
        out_specs=pl.BlockSpec((pt // 128, 128), lambda i: (i, 0)),
        out_shape=jax.ShapeDtypeStruct((P // 128, 128), jnp.int32),
    )(points)


def _mlp_table(h2, w1, b1, w2p, b2p, vt):
    C, V = h2.shape
    H = w1.shape[0]
    OP = w2p.shape[0]

    def body(h_ref, w1_ref, b1_ref, w2_ref, b2_ref, out_ref):
        blk = h_ref[...].astype(jnp.bfloat16)
        x1 = lax.dot_general(
            blk, w1_ref[...].astype(jnp.bfloat16), (((0,), (1,)), ((), ())),
            preferred_element_type=jnp.float32)
        x1 = jnp.maximum(x1 + b1_ref[...], 0.0).astype(jnp.bfloat16)
        x2 = lax.dot_general(
            x1, w2_ref[...].astype(jnp.bfloat16), (((1,), (1,)), ((), ())),
            preferred_element_type=jnp.float32)
        out_ref[...] = x2 + b2_ref[...]

    return pl.pallas_call(
        body,
        grid=(V // vt,),
        in_specs=[
            pl.BlockSpec((C, vt), lambda i: (0, i)),
            pl.BlockSpec((H, C), lambda i: (0, 0)),
            pl.BlockSpec((1, H), lambda i: (0, 0)),
            pl.BlockSpec((OP, H), lambda i: (0, 0)),
            pl.BlockSpec((1, OP), lambda i: (0, 0)),
        ],
        out_specs=pl.BlockSpec((vt, OP), lambda i: (i, 0)),
        out_shape=jax.ShapeDtypeStruct((V, OP), jnp.float32),
    )(h2, w1, b1, w2p, b2p)


def _gather_rows(table, idx2d, n_out):
    V, OP = table.shape
    NR, G = idx2d.shape
    P = NR * G
    npw = P // _NW
    nchunk = npw // G
    mesh = plsc.VectorSubcoreMesh(core_axis_name="c", subcore_axis_name="s")

    @functools.partial(
        pl.kernel,
        mesh=mesh,
        compiler_params=pltpu.CompilerParams(use_tc_tiling_on_sc=False),
        out_type=jax.ShapeDtypeStruct((P, n_out), jnp.float32),
        scratch_types=[
            pltpu.VMEM((nchunk, G), jnp.int32),
            pltpu.VMEM((2, G, OP), jnp.float32),
            pltpu.SemaphoreType.DMA,
            pltpu.SemaphoreType.DMA,
        ],
    )
    def body(table_hbm, idx_hbm, out_hbm, idx_v, row_v, sem0, sem1):
        wid = lax.axis_index("s") * _NC + lax.axis_index("c")
        base = wid * npw
        pltpu.sync_copy(idx_hbm.at[pl.ds(wid * nchunk, nchunk)], idx_v)

        sems = (sem0, sem1)

        def start_gather(j, b):
            pltpu.async_copy(
                table_hbm.at[idx_v.at[j]], row_v.at[b], sems[b])

        def drain_write(j, b):
            pltpu.make_async_copy(
                table_hbm.at[idx_v.at[j]], row_v.at[b], sems[b]).wait()
            pltpu.sync_copy(
                row_v.at[b, :, pl.ds(0, n_out)],
                out_hbm.at[pl.ds(base + j * G, G)])

        start_gather(0, 0)

        def gather_body(jj, carry):
            j0 = jj * 2

            @pl.when(j0 + 1 < nchunk)
            def _():
                start_gather(j0 + 1, 1)

            drain_write(j0, 0)

            @pl.when(j0 + 1 < nchunk)
            def _():
                @pl.when(j0 + 2 < nchunk)
                def _():
                    start_gather(j0 + 2, 0)

                drain_write(j0 + 1, 1)

            return carry

        lax.fori_loop(0, (nchunk + 1) // 2, gather_body, 0)

    return body(table, idx2d)


def kernel(h, points, W1, b1, W2, b2):
    C = h.shape[1]
    d0, d1, d2 = h.shape[2], h.shape[3], h.shape[4]
    V = d0 * d1 * d2
    P = points.shape[1]
    H = W1.shape[0]
    OUT = W2.shape[0]
    OP = 128

    h2 = h.reshape(C, V)
    w2p = jnp.zeros((OP, H), W2.dtype).at[:OUT, :].set(W2)
    b2p = jnp.zeros((OP,), b2.dtype).at[:OUT].set(b2)

    idx2d = _point_indices(points, d1 * d2, d2)
    table = _mlp_table(h2, W1, b1.reshape(1, H), w2p, b2p.reshape(1, OP), 2048)
    n_out = ((OUT + 7) // 8) * 8
    out = _gather_rows(table, idx2d, n_out)
    return out[:, :OUT]


# speedup vs baseline: 1.4333x; 1.1554x over previous
import functools

import jax
import jax.numpy as jnp
from jax import lax
from jax.experimental import pallas as pl
from jax.experimental.pallas import tpu as pltpu
from jax.experimental.pallas import tpu_sc as plsc

_NC = 2
_NS = 16
_NW = _NC * _NS
_LANES = 16
_G = 128


def _mlp_table(h2, w1, b1, w2p, b2p, vt):
    C, V = h2.shape
    H = w1.shape[0]
    OP = w2p.shape[0]

    def body(h_ref, w1_ref, b1_ref, w2_ref, b2_ref, out_ref):
        blk = h_ref[...]
        x1 = lax.dot_general(
            blk, w1_ref[...].astype(jnp.bfloat16), (((0,), (1,)), ((), ())),
            preferred_element_type=jnp.float32)
        x1 = jnp.maximum(x1 + b1_ref[...], 0.0).astype(jnp.bfloat16)
        x2 = lax.dot_general(
            x1, w2_ref[...].astype(jnp.bfloat16), (((1,), (1,)), ((), ())),
            preferred_element_type=jnp.float32)
        out_ref[...] = x2 + b2_ref[...]

    return pl.pallas_call(
        body,
        grid=(V // vt,),
        in_specs=[
            pl.BlockSpec((C, vt), lambda i: (0, i)),
            pl.BlockSpec((H, C), lambda i: (0, 0)),
            pl.BlockSpec((1, H), lambda i: (0, 0)),
            pl.BlockSpec((OP, H), lambda i: (0, 0)),
            pl.BlockSpec((1, OP), lambda i: (0, 0)),
        ],
        out_specs=pl.BlockSpec((vt, OP), lambda i: (i, 0)),
        out_shape=jax.ShapeDtypeStruct((V, OP), jnp.float32),
    )(h2, w1, b1, w2p, b2p)


def _gather_rows(table, xs, ys, zs, s0, s1, n_out):
    V, OP = table.shape
    P = xs.shape[0]
    npw = P // _NW
    nchunk = npw // _G
    mesh = plsc.VectorSubcoreMesh(core_axis_name="c", subcore_axis_name="s")

    @functools.partial(
        pl.kernel,
        mesh=mesh,
        compiler_params=pltpu.CompilerParams(use_tc_tiling_on_sc=False),
        out_type=jax.ShapeDtypeStruct((P, n_out), jnp.float32),
        scratch_types=[
            pltpu.VMEM((npw,), jnp.float32),
            pltpu.VMEM((npw,), jnp.float32),
            pltpu.VMEM((npw,), jnp.float32),
            pltpu.VMEM((npw,), jnp.int32),
            pltpu.VMEM((2, _G, OP), jnp.float32),
            pltpu.SemaphoreType.DMA,
            pltpu.SemaphoreType.DMA,
        ],
    )
    def body(table_hbm, xs_hbm, ys_hbm, zs_hbm, out_hbm,
             x_v, y_v, z_v, idx_v, row_v, sem0, sem1):
        wid = lax.axis_index("s") * _NC + lax.axis_index("c")
        base = wid * npw
        pltpu.sync_copy(xs_hbm.at[pl.ds(base, npw)], x_v)
        pltpu.sync_copy(ys_hbm.at[pl.ds(base, npw)], y_v)
        pltpu.sync_copy(zs_hbm.at[pl.ds(base, npw)], z_v)

        def idx_body(i, carry):
            sl = pl.ds(i * _LANES, _LANES)
            ix = x_v[sl].astype(jnp.int32)
            iy = y_v[sl].astype(jnp.int32)
            iz = z_v[sl].astype(jnp.int32)
            idx_v[sl] = ix * s0 + iy * s1 + iz
            return carry

        lax.fori_loop(0, npw // _LANES, idx_body, 0)

        sems = (sem0, sem1)

        def start_gather(j, b):
            pltpu.async_copy(
                table_hbm.at[idx_v.at[pl.ds(j * _G, _G)]], row_v.at[b],
                sems[b])

        def drain_write(j, b):
            pltpu.make_async_copy(
                table_hbm.at[idx_v.at[pl.ds(j * _G, _G)]], row_v.at[b],
                sems[b]).wait()
            pltpu.sync_copy(
                row_v.at[b, :, pl.ds(0, n_out)],
                out_hbm.at[pl.ds(base + j * _G, _G)])

        start_gather(0, 0)

        def gather_body(jj, carry):
            j0 = jj * 2

            @pl.when(j0 + 1 < nchunk)
            def _():
                start_gather(j0 + 1, 1)

            drain_write(j0, 0)

            @pl.when(j0 + 1 < nchunk)
            def _():
                @pl.when(j0 + 2 < nchunk)
                def _():
                    start_gather(j0 + 2, 0)

                drain_write(j0 + 1, 1)

            return carry

        lax.fori_loop(0, (nchunk + 1) // 2, gather_body, 0)

    return body(table, xs, ys, zs)


def kernel(h, points, W1, b1, W2, b2):
    C = h.shape[1]
    d0, d1, d2 = h.shape[2], h.shape[3], h.shape[4]
    V = d0 * d1 * d2
    P = points.shape[1]
    H = W1.shape[0]
    OUT = W2.shape[0]
    OP = 128

    h2 = h.astype(jnp.bfloat16).reshape(C, V)
    pts = points.reshape(P, 3).T
    xs, ys, zs = pts[0], pts[1], pts[2]

    w2p = jnp.zeros((OP, H), W2.dtype).at[:OUT, :].set(W2)
    b2p = jnp.zeros((OP,), b2.dtype).at[:OUT].set(b2)

    table = _mlp_table(h2, W1, b1.reshape(1, H), w2p, b2p.reshape(1, OP), 2048)
    n_out = ((OUT + 7) // 8) * 8
    out = _gather_rows(table, xs, ys, zs, d1 * d2, d2, n_out)
    return out[:, :OUT]


# speedup vs baseline: 1.4971x; 1.0445x over previous
import functools

import jax
import jax.numpy as jnp
from jax import lax
from jax.experimental import pallas as pl
from jax.experimental.pallas import tpu as pltpu
from jax.experimental.pallas import tpu_sc as plsc

_NC = 2
_NS = 16
_NW = _NC * _NS
_LANES = 16
_G = 128


def _mlp_table(h2, w1, b1, w2p, b2p, vt):
    C, V = h2.shape
    H = w1.shape[0]
    OP = w2p.shape[0]

    def body(h_ref, w1_ref, b1_ref, w2_ref, b2_ref, out_ref):
        blk = h_ref[...].astype(jnp.bfloat16)
        x1 = lax.dot_general(
            blk, w1_ref[...].astype(jnp.bfloat16), (((0,), (1,)), ((), ())),
            preferred_element_type=jnp.float32)
        x1 = jnp.maximum(x1 + b1_ref[...], 0.0).astype(jnp.bfloat16)
        x2 = lax.dot_general(
            x1, w2_ref[...].astype(jnp.bfloat16), (((1,), (1,)), ((), ())),
            preferred_element_type=jnp.float32)
        out_ref[...] = x2 + b2_ref[...]

    return pl.pallas_call(
        body,
        grid=(V // vt,),
        in_specs=[
            pl.BlockSpec((C, vt), lambda i: (0, i)),
            pl.BlockSpec((H, C), lambda i: (0, 0)),
            pl.BlockSpec((1, H), lambda i: (0, 0)),
            pl.BlockSpec((OP, H), lambda i: (0, 0)),
            pl.BlockSpec((1, OP), lambda i: (0, 0)),
        ],
        out_specs=pl.BlockSpec((vt, OP), lambda i: (i, 0)),
        out_shape=jax.ShapeDtypeStruct((V, OP), jnp.float32),
    )(h2, w1, b1, w2p, b2p)


def _gather_rows(table, xs, ys, zs, s0, s1, n_out):
    V, OP = table.shape
    P = xs.shape[0]
    npw = P // _NW
    nchunk = npw // _G
    mesh = plsc.VectorSubcoreMesh(core_axis_name="c", subcore_axis_name="s")

    @functools.partial(
        pl.kernel,
        mesh=mesh,
        compiler_params=pltpu.CompilerParams(use_tc_tiling_on_sc=False),
        out_type=jax.ShapeDtypeStruct((P, n_out), jnp.float32),
        scratch_types=[
            pltpu.VMEM((npw,), jnp.float32),
            pltpu.VMEM((npw,), jnp.float32),
            pltpu.VMEM((npw,), jnp.float32),
            pltpu.VMEM((npw,), jnp.int32),
            pltpu.VMEM((4, _G, OP), jnp.float32),
            pltpu.SemaphoreType.DMA,
            pltpu.SemaphoreType.DMA,
            pltpu.SemaphoreType.DMA,
            pltpu.SemaphoreType.DMA,
        ],
    )
    def body(table_hbm, xs_hbm, ys_hbm, zs_hbm, out_hbm,
             x_v, y_v, z_v, idx_v, row_v, sem0, sem1, sem2, sem3):
        wid = lax.axis_index("s") * _NC + lax.axis_index("c")
        base = wid * npw
        pltpu.sync_copy(xs_hbm.at[pl.ds(base, npw)], x_v)
        pltpu.sync_copy(ys_hbm.at[pl.ds(base, npw)], y_v)
        pltpu.sync_copy(zs_hbm.at[pl.ds(base, npw)], z_v)

        def idx_body(i, carry):
            sl = pl.ds(i * _LANES, _LANES)
            ix = x_v[sl].astype(jnp.int32)
            iy = y_v[sl].astype(jnp.int32)
            iz = z_v[sl].astype(jnp.int32)
            idx_v[sl] = ix * s0 + iy * s1 + iz
            return carry

        lax.fori_loop(0, npw // _LANES, idx_body, 0)

        sems = (sem0, sem1, sem2, sem3)
        nbuf = 4

        def start_gather(j, b):
            pltpu.async_copy(
                table_hbm.at[idx_v.at[pl.ds(j * _G, _G)]], row_v.at[b],
                sems[b])

        def drain_write(j, b):
            pltpu.make_async_copy(
                table_hbm.at[idx_v.at[pl.ds(j * _G, _G)]], row_v.at[b],
                sems[b]).wait()
            pltpu.sync_copy(
                row_v.at[b, :, pl.ds(0, n_out)],
                out_hbm.at[pl.ds(base + j * _G, _G)])

        for b in range(nbuf):
            start_gather(b, b)

        def gather_body(jj, carry):
            j0 = jj * nbuf
            for b in range(nbuf):
                drain_write(j0 + b, b)

                @pl.when(j0 + b + nbuf < nchunk)
                def _():
                    start_gather(j0 + b + nbuf, b)

            return carry

        lax.fori_loop(0, nchunk // nbuf, gather_body, 0)

    return body(table, xs, ys, zs)


def kernel(h, points, W1, b1, W2, b2):
    C = h.shape[1]
    d0, d1, d2 = h.shape[2], h.shape[3], h.shape[4]
    V = d0 * d1 * d2
    P = points.shape[1]
    H = W1.shape[0]
    OUT = W2.shape[0]
    OP = 128

    h2 = h.reshape(C, V)
    pts = points.reshape(P, 3).T
    xs, ys, zs = pts[0], pts[1], pts[2]

    w2p = jnp.zeros((OP, H), W2.dtype).at[:OUT, :].set(W2)
    b2p = jnp.zeros((OP,), b2.dtype).at[:OUT].set(b2)

    table = _mlp_table(h2, W1, b1.reshape(1, H), w2p, b2p.reshape(1, OP), 2048)
    n_out = ((OUT + 7) // 8) * 8
    out = _gather_rows(table, xs, ys, zs, d1 * d2, d2, n_out)
    return out[:, :OUT]


# speedup vs baseline: 1.6651x; 1.1122x over previous
import functools

import jax
import jax.numpy as jnp
from jax import lax
from jax.experimental import pallas as pl
from jax.experimental.pallas import tpu as pltpu
from jax.experimental.pallas import tpu_sc as plsc

_NC = 2
_NS = 16
_NW = _NC * _NS
_LANES = 16
_G = 128


def _mlp_table(h2, w1, b1, w2p, b2p, vt):
    C, V = h2.shape
    H = w1.shape[0]
    OP = w2p.shape[0]

    def body(h_ref, w1_ref, b1_ref, w2_ref, b2_ref, out_ref):
        blk = h_ref[...].astype(jnp.bfloat16)
        x1 = lax.dot_general(
            blk, w1_ref[...].astype(jnp.bfloat16), (((0,), (1,)), ((), ())),
            preferred_element_type=jnp.float32)
        x1 = jnp.maximum(x1 + b1_ref[...], 0.0).astype(jnp.bfloat16)
        x2 = lax.dot_general(
            x1, w2_ref[...].astype(jnp.bfloat16), (((1,), (1,)), ((), ())),
            preferred_element_type=jnp.float32)
        out_ref[...] = x2 + b2_ref[...]

    return pl.pallas_call(
        body,
        grid=(V // vt,),
        in_specs=[
            pl.BlockSpec((C, vt), lambda i: (0, i)),
            pl.BlockSpec((H, C), lambda i: (0, 0)),
            pl.BlockSpec((1, H), lambda i: (0, 0)),
            pl.BlockSpec((OP, H), lambda i: (0, 0)),
            pl.BlockSpec((1, OP), lambda i: (0, 0)),
        ],
        out_specs=pl.BlockSpec((vt, OP), lambda i: (i, 0)),
        out_shape=jax.ShapeDtypeStruct((V, OP), jnp.float32),
    )(h2, w1, b1, w2p, b2p)


def _gather_rows(table, xs, ys, zs, s0, s1, n_out):
    V, OP = table.shape
    P = xs.shape[0]
    npw = P // _NW
    nchunk = npw // _G
    mesh = plsc.VectorSubcoreMesh(core_axis_name="c", subcore_axis_name="s")

    @functools.partial(
        pl.kernel,
        mesh=mesh,
        compiler_params=pltpu.CompilerParams(use_tc_tiling_on_sc=False),
        out_type=jax.ShapeDtypeStruct((P, n_out), jnp.float32),
        scratch_types=[
            pltpu.VMEM((npw,), jnp.float32),
            pltpu.VMEM((npw,), jnp.float32),
            pltpu.VMEM((npw,), jnp.float32),
            pltpu.VMEM((npw,), jnp.int32),
            pltpu.VMEM((4, _G, OP), jnp.float32),
            pltpu.SemaphoreType.DMA,
            pltpu.SemaphoreType.DMA,
            pltpu.SemaphoreType.DMA,
            pltpu.SemaphoreType.DMA,
        ],
    )
    def body(table_hbm, xs_hbm, ys_hbm, zs_hbm, out_hbm,
             x_v, y_v, z_v, idx_v, row_v, sem0, sem1, sem2, sem3):
        wid = lax.axis_index("s") * _NC + lax.axis_index("c")
        base = wid * npw
        pltpu.sync_copy(xs_hbm.at[pl.ds(base, npw)], x_v)
        pltpu.sync_copy(ys_hbm.at[pl.ds(base, npw)], y_v)
        pltpu.sync_copy(zs_hbm.at[pl.ds(base, npw)], z_v)

        def idx_body(i, carry):
            sl = pl.ds(i * _LANES, _LANES)
            ix = x_v[sl].astype(jnp.int32)
            iy = y_v[sl].astype(jnp.int32)
            iz = z_v[sl].astype(jnp.int32)
            idx_v[sl] = ix * s0 + iy * s1 + iz
            return carry

        lax.fori_loop(0, npw // _LANES, idx_body, 0)

        sems = (sem0, sem1, sem2, sem3)
        nbuf = 4

        def start_gather(j, b):
            pltpu.async_copy(
                table_hbm.at[idx_v.at[pl.ds(j * _G, _G)]], row_v.at[b],
                sems[b])

        def drain_write(j, b):
            pltpu.make_async_copy(
                table_hbm.at[idx_v.at[pl.ds(j * _G, _G)]], row_v.at[b],
                sems[b]).wait()
            pltpu.sync_copy(
                row_v.at[b, :, pl.ds(0, n_out)],
                out_hbm.at[pl.ds(base + j * _G, _G)])

        for b in range(nbuf):
            start_gather(b, b)

        def gather_body(jj, carry):
            j0 = jj * nbuf
            for b in range(nbuf):
                drain_write(j0 + b, b)

                @pl.when(j0 + b + nbuf < nchunk)
                def _():
                    start_gather(j0 + b + nbuf, b)

            return carry

        lax.fori_loop(0, nchunk // nbuf, gather_body, 0)

    return body(table, xs, ys, zs)


def kernel(h, points, W1, b1, W2, b2):
    C = h.shape[1]
    d0, d1, d2 = h.shape[2], h.shape[3], h.shape[4]
    V = d0 * d1 * d2
    P = points.shape[1]
    H = W1.shape[0]
    OUT = W2.shape[0]
    OP = 128

    h2 = h.reshape(C, V)
    pts = points.reshape(P, 3).T
    xs, ys, zs = pts[0], pts[1], pts[2]

    w2p = jnp.zeros((OP, H), W2.dtype).at[:OUT, :].set(W2)
    b2p = jnp.zeros((OP,), b2.dtype).at[:OUT].set(b2)

    table = _mlp_table(h2, W1, b1.reshape(1, H), w2p, b2p.reshape(1, OP), 4096)
    n_out = ((OUT + 7) // 8) * 8
    out = _gather_rows(table, xs, ys, zs, d1 * d2, d2, n_out)
    return out[:, :OUT]


# speedup vs baseline: 1.7567x; 1.0550x over previous
import functools

import jax
import jax.numpy as jnp
from jax import lax
from jax.experimental import pallas as pl
from jax.experimental.pallas import tpu as pltpu
from jax.experimental.pallas import tpu_sc as plsc

_NC = 2
_NS = 16
_NW = _NC * _NS
_LANES = 16
_G = 128


def _mlp_table(h2, w1, b1, w2p, b2p, vt):
    C, V = h2.shape
    H = w1.shape[0]
    OP = w2p.shape[0]

    def body(h_ref, w1_ref, b1_ref, w2_ref, b2_ref, out_ref):
        blk = h_ref[...].astype(jnp.bfloat16)
        x1 = lax.dot_general(
            blk, w1_ref[...].astype(jnp.bfloat16), (((0,), (1,)), ((), ())),
            preferred_element_type=jnp.float32)
        x1 = jnp.maximum(x1 + b1_ref[...], 0.0).astype(jnp.bfloat16)
        x2 = lax.dot_general(
            x1, w2_ref[...].astype(jnp.bfloat16), (((1,), (1,)), ((), ())),
            preferred_element_type=jnp.float32)
        out_ref[...] = x2 + b2_ref[...]

    return pl.pallas_call(
        body,
        grid=(V // vt,),
        in_specs=[
            pl.BlockSpec((C, vt), lambda i: (0, i)),
            pl.BlockSpec((H, C), lambda i: (0, 0)),
            pl.BlockSpec((1, H), lambda i: (0, 0)),
            pl.BlockSpec((OP, H), lambda i: (0, 0)),
            pl.BlockSpec((1, OP), lambda i: (0, 0)),
        ],
        out_specs=pl.BlockSpec((vt, OP), lambda i: (i, 0)),
        out_shape=jax.ShapeDtypeStruct((V, OP), jnp.float32),
    )(h2, w1, b1, w2p, b2p)


def _gather_rows(table, xs, ys, zs, s0, s1, n_out):
    V, OP = table.shape
    P = xs.shape[0]
    npw = P // _NW
    nchunk = npw // _G
    mesh = plsc.VectorSubcoreMesh(core_axis_name="c", subcore_axis_name="s")

    @functools.partial(
        pl.kernel,
        mesh=mesh,
        compiler_params=pltpu.CompilerParams(use_tc_tiling_on_sc=False),
        out_type=jax.ShapeDtypeStruct((P, n_out), jnp.float32),
        scratch_types=[
            pltpu.VMEM((npw,), jnp.float32),
            pltpu.VMEM((npw,), jnp.float32),
            pltpu.VMEM((npw,), jnp.float32),
            pltpu.VMEM((npw,), jnp.int32),
            pltpu.VMEM((4, _G, OP), jnp.float32),
            pltpu.SemaphoreType.DMA,
            pltpu.SemaphoreType.DMA,
            pltpu.SemaphoreType.DMA,
            pltpu.SemaphoreType.DMA,
        ],
    )
    def body(table_hbm, xs_hbm, ys_hbm, zs_hbm, out_hbm,
             x_v, y_v, z_v, idx_v, row_v, sem0, sem1, sem2, sem3):
        wid = lax.axis_index("s") * _NC + lax.axis_index("c")
        base = wid * npw
        pltpu.sync_copy(xs_hbm.at[pl.ds(base, npw)], x_v)
        pltpu.sync_copy(ys_hbm.at[pl.ds(base, npw)], y_v)
        pltpu.sync_copy(zs_hbm.at[pl.ds(base, npw)], z_v)

        def idx_body(i, carry):
            sl = pl.ds(i * _LANES, _LANES)
            ix = x_v[sl].astype(jnp.int32)
            iy = y_v[sl].astype(jnp.int32)
            iz = z_v[sl].astype(jnp.int32)
            idx_v[sl] = ix * s0 + iy * s1 + iz
            return carry

        lax.fori_loop(0, npw // _LANES, idx_body, 0)

        sems = (sem0, sem1, sem2, sem3)
        nbuf = 4

        def start_gather(j, b):
            pltpu.async_copy(
                table_hbm.at[idx_v.at[pl.ds(j * _G, _G)]], row_v.at[b],
                sems[b])

        def drain_write(j, b):
            pltpu.make_async_copy(
                table_hbm.at[idx_v.at[pl.ds(j * _G, _G)]], row_v.at[b],
                sems[b]).wait()
            pltpu.sync_copy(
                row_v.at[b, :, pl.ds(0, n_out)],
                out_hbm.at[pl.ds(base + j * _G, _G)])

        for b in range(nbuf):
            start_gather(b, b)

        def gather_body(jj, carry):
            j0 = jj * nbuf
            for b in range(nbuf):
                drain_write(j0 + b, b)

                @pl.when(j0 + b + nbuf < nchunk)
                def _():
                    start_gather(j0 + b + nbuf, b)

            return carry

        lax.fori_loop(0, nchunk // nbuf, gather_body, 0)

    return body(table, xs, ys, zs)


def kernel(h, points, W1, b1, W2, b2):
    C = h.shape[1]
    d0, d1, d2 = h.shape[2], h.shape[3], h.shape[4]
    V = d0 * d1 * d2
    P = points.shape[1]
    H = W1.shape[0]
    OUT = W2.shape[0]
    OP = 128

    h2 = h.reshape(C, V)
    pts = points.reshape(P, 3).T
    xs, ys, zs = pts[0], pts[1], pts[2]

    w2p = jnp.zeros((OP, H), W2.dtype).at[:OUT, :].set(W2)
    b2p = jnp.zeros((OP,), b2.dtype).at[:OUT].set(b2)

    table = _mlp_table(h2, W1, b1.reshape(1, H), w2p, b2p.reshape(1, OP), 8192)
    n_out = ((OUT + 7) // 8) * 8
    out = _gather_rows(table, xs, ys, zs, d1 * d2, d2, n_out)
    return out[:, :OUT]


# speedup vs baseline: 1.7659x; 1.0052x over previous
import functools

import jax
import jax.numpy as jnp
from jax import lax
from jax.experimental import pallas as pl
from jax.experimental.pallas import tpu as pltpu
from jax.experimental.pallas import tpu_sc as plsc

_NC = 2
_NS = 16
_NW = _NC * _NS
_LANES = 16
_G = 128


def _mlp_table(h2, w1, b1, w2p, b2p, vt):
    C, V = h2.shape
    H = w1.shape[0]
    OP = w2p.shape[0]

    def body(h_ref, w1_ref, b1_ref, w2_ref, b2_ref, out_ref):
        blk = h_ref[...].astype(jnp.bfloat16)
        x1 = lax.dot_general(
            blk, w1_ref[...].astype(jnp.bfloat16), (((0,), (1,)), ((), ())),
            preferred_element_type=jnp.float32)
        x1 = jnp.maximum(x1 + b1_ref[...], 0.0).astype(jnp.bfloat16)
        x2 = lax.dot_general(
            x1, w2_ref[...].astype(jnp.bfloat16), (((1,), (1,)), ((), ())),
            preferred_element_type=jnp.float32)
        out_ref[...] = x2 + b2_ref[...]

    return pl.pallas_call(
        body,
        grid=(V // vt,),
        in_specs=[
            pl.BlockSpec((C, vt), lambda i: (0, i)),
            pl.BlockSpec((H, C), lambda i: (0, 0)),
            pl.BlockSpec((1, H), lambda i: (0, 0)),
            pl.BlockSpec((OP, H), lambda i: (0, 0)),
            pl.BlockSpec((1, OP), lambda i: (0, 0)),
        ],
        out_specs=pl.BlockSpec((vt, OP), lambda i: (i, 0)),
        out_shape=jax.ShapeDtypeStruct((V, OP), jnp.float32),
    )(h2, w1, b1, w2p, b2p)


def _gather_rows(table, xs, ys, zs, s0, s1, n_out):
    V, OP = table.shape
    P = xs.shape[0]
    npw = P // _NW
    nchunk = npw // _G
    mesh = plsc.VectorSubcoreMesh(core_axis_name="c", subcore_axis_name="s")

    @functools.partial(
        pl.kernel,
        mesh=mesh,
        compiler_params=pltpu.CompilerParams(use_tc_tiling_on_sc=False),
        out_type=jax.ShapeDtypeStruct((P, n_out), jnp.float32),
        scratch_types=[
            pltpu.VMEM((npw,), jnp.float32),
            pltpu.VMEM((npw,), jnp.float32),
            pltpu.VMEM((npw,), jnp.float32),
            pltpu.VMEM((npw,), jnp.int32),
            pltpu.VMEM((4, _G, OP), jnp.float32),
            pltpu.SemaphoreType.DMA,
            pltpu.SemaphoreType.DMA,
            pltpu.SemaphoreType.DMA,
            pltpu.SemaphoreType.DMA,
        ],
    )
    def body(table_hbm, xs_hbm, ys_hbm, zs_hbm, out_hbm,
             x_v, y_v, z_v, idx_v, row_v, sem0, sem1, sem2, sem3):
        wid = lax.axis_index("s") * _NC + lax.axis_index("c")
        base = wid * npw
        pltpu.sync_copy(xs_hbm.at[pl.ds(base, npw)], x_v)
        pltpu.sync_copy(ys_hbm.at[pl.ds(base, npw)], y_v)
        pltpu.sync_copy(zs_hbm.at[pl.ds(base, npw)], z_v)

        def idx_body(i, carry):
            sl = pl.ds(i * _LANES, _LANES)
            ix = x_v[sl].astype(jnp.int32)
            iy = y_v[sl].astype(jnp.int32)
            iz = z_v[sl].astype(jnp.int32)
            idx_v[sl] = ix * s0 + iy * s1 + iz
            return carry

        lax.fori_loop(0, npw // _LANES, idx_body, 0)

        sems = (sem0, sem1, sem2, sem3)
        nbuf = 4

        def start_gather(j, b):
            pltpu.async_copy(
                table_hbm.at[idx_v.at[pl.ds(j * _G, _G)]], row_v.at[b],
                sems[b])

        def drain_write(j, b):
            pltpu.make_async_copy(
                table_hbm.at[idx_v.at[pl.ds(j * _G, _G)]], row_v.at[b],
                sems[b]).wait()
            pltpu.sync_copy(
                row_v.at[b, :, pl.ds(0, n_out)],
                out_hbm.at[pl.ds(base + j * _G, _G)])

        for b in range(nbuf):
            start_gather(b, b)

        def gather_body(jj, carry):
            j0 = jj * nbuf
            for b in range(nbuf):
                drain_write(j0 + b, b)

                @pl.when(j0 + b + nbuf < nchunk)
                def _():
                    start_gather(j0 + b + nbuf, b)

            return carry

        lax.fori_loop(0, nchunk // nbuf, gather_body, 0)

    return body(table, xs, ys, zs)


def kernel(h, points, W1, b1, W2, b2):
    C = h.shape[1]
    d0, d1, d2 = h.shape[2], h.shape[3], h.shape[4]
    V = d0 * d1 * d2
    P = points.shape[1]
    H = W1.shape[0]
    OUT = W2.shape[0]
    OP = 128

    h2 = h.reshape(C, V)
    pts = points.reshape(P, 3).T
    xs, ys, zs = pts[0], pts[1], pts[2]

    w2p = jnp.zeros((OP, H), W2.dtype).at[:OUT, :].set(W2)
    b2p = jnp.zeros((OP,), b2.dtype).at[:OUT].set(b2)

    table = _mlp_table(h2, W1, b1.reshape(1, H), w2p, b2p.reshape(1, OP), 16384)
    n_out = ((OUT + 7) // 8) * 8
    out = _gather_rows(table, xs, ys, zs, d1 * d2, d2, n_out)
    return out[:, :OUT]
